# baseline (speedup 1.0000x reference)
import functools

import jax
import jax.numpy as jnp
from jax import lax
from jax.experimental import pallas as pl
from jax.experimental.pallas import tpu as pltpu
from jax.experimental.pallas import tpu_sc as plsc

B, SL, DL, D = 1024, 200, 200, 64
L2 = SL + DL
NP = L2 // 2
LP = 512
NPP = LP // 2
NC, NS = 2, 16
NW = NC * NS
ROWS_PER_W = B // NW
HSIZE = 100000
RB = 8

_mesh = plsc.VectorSubcoreMesh(core_axis_name="c", subcore_axis_name="s",
                               num_cores=NC, num_subcores=NS)


def _sc_count_body(ids_hbm, wvec_hbm, zvec_hbm, c0_hbm, c1_hbm,
                   hist, ids_v, wvec_v, zvec_v, idx_v, cnt_v, c0_v, c1_v):
    c = lax.axis_index("c")
    s = lax.axis_index("s")
    wid = c * NS + s
    pltpu.sync_copy(wvec_hbm, wvec_v)
    pltpu.sync_copy(zvec_hbm, zvec_v)

    def row_step(r, carry):
        row = wid * ROWS_PER_W + r
        pltpu.sync_copy(ids_hbm.at[row], ids_v)
        for j in range(4):
            for k in range(8):
                sl = pl.ds(k * 16, 16)
                idx_v[j, sl] = ids_v[j, sl] + s * HSIZE
        for j in range(4):
            pltpu.sync_copy(zvec_v.at[j], hist.at[idx_v.at[j]])
        for j in range(4):
            pltpu.sync_copy(wvec_v.at[j], hist.at[idx_v.at[j]], add=True)
        for j in range(4):
            pltpu.sync_copy(hist.at[idx_v.at[j]], cnt_v.at[j])
        for j in range(4):
            for k in range(8):
                sl = pl.ds(k * 16, 16)
                ids16 = ids_v[j, sl]
                cnt16 = cnt_v[j, sl]
                nz = ids16 != 0
                c0_v[j, sl] = jnp.where(nz, cnt16 & 0xFFFF, 0).astype(jnp.float32)
                c1_v[j, sl] = jnp.where(nz, cnt16 >> 16, 0).astype(jnp.float32)
        pltpu.sync_copy(c0_v, c0_hbm.at[row])
        pltpu.sync_copy(c1_v, c1_hbm.at[row])
        return carry

    lax.fori_loop(0, ROWS_PER_W, row_step, 0)


@functools.partial(
    pl.kernel,
    out_type=(
        jax.ShapeDtypeStruct((B, 4, 128), jnp.float32),
        jax.ShapeDtypeStruct((B, 4, 128), jnp.float32),
    ),
    mesh=_mesh,
    scratch_types=[
        pltpu.VMEM_SHARED((NS * HSIZE,), jnp.int32),
        pltpu.VMEM((4, 128), jnp.int32),
        pltpu.VMEM((4, 128), jnp.int32),
        pltpu.VMEM((4, 128), jnp.int32),
        pltpu.VMEM((4, 128), jnp.int32),
        pltpu.VMEM((4, 128), jnp.int32),
        pltpu.VMEM((4, 128), jnp.float32),
        pltpu.VMEM((4, 128), jnp.float32),
    ],
)
def _sc_count(*args):
    _sc_count_body(*args)


def _tc_encode_body(c0_ref, c1_ref, w01_ref, w2_ref, b2r_ref,
                    src_out, dst_out):
    w01 = w01_ref[...]
    w2 = w2_ref[...]
    b2r = b2r_ref[...]
    one_row = jnp.ones((1, LP), jnp.float32)
    for b in range(RB):
        s0 = jnp.concatenate([c0_ref[b:b + 1, :], one_row], axis=0)
        s1 = jnp.concatenate([c1_ref[b:b + 1, :], one_row], axis=0)
        p0t = jnp.dot(w01, s0, preferred_element_type=jnp.float32)
        p1t = jnp.dot(w01, s1, preferred_element_type=jnp.float32)
        ht = jnp.maximum(p0t, 0.0) + jnp.maximum(p1t, 0.0)
        fe = lax.dot_general(ht[:, :LP // 2], w2, (((0,), (0,)), ((), ())),
                             preferred_element_type=jnp.float32)
        fo = lax.dot_general(ht[:, LP // 2:], w2, (((0,), (0,)), ((), ())),
                             preferred_element_type=jnp.float32)
        featp = jnp.concatenate([fe, fo], axis=1) + b2r
        src_out[b, :, :] = featp[:NP // 2, :]
        dst_out[b, :, :] = featp[NP // 2:NP, :]


def _tc_encode(c0, c1, w01, w2, b2r):
    return pl.pallas_call(
        _tc_encode_body,
        grid=(B // RB,),
        in_specs=[
            pl.BlockSpec((RB, LP), lambda i: (i, 0)),
            pl.BlockSpec((RB, LP), lambda i: (i, 0)),
            pl.BlockSpec((D, 2), lambda i: (0, 0)),
            pl.BlockSpec((D, D), lambda i: (0, 0)),
            pl.BlockSpec((1, 2 * D), lambda i: (0, 0)),
        ],
        out_specs=[
            pl.BlockSpec((RB, NP // 2, 128), lambda i: (i, 0, 0)),
            pl.BlockSpec((RB, NP // 2, 128), lambda i: (i, 0, 0)),
        ],
        out_shape=[
            jax.ShapeDtypeStruct((B, NP // 2, 128), jnp.float32),
            jax.ShapeDtypeStruct((B, NP // 2, 128), jnp.float32),
        ],
    )(c0, c1, w01, w2, b2r)


@jax.jit
def kernel(src_ids, dst_ids, W1, b1, W2, b2):
    ids = jnp.concatenate([src_ids.astype(jnp.int32),
                           dst_ids.astype(jnp.int32)], axis=1)
    idsp = jnp.pad(ids, ((0, 0), (0, LP - L2)))
    ids_eo = jnp.concatenate([idsp[:, 0::2], idsp[:, 1::2]], axis=1)
    ids_pad = ids_eo.reshape(B, 4, 128)
    half = jnp.concatenate([
        jnp.full((SL // 2,), 1, jnp.int32),
        jnp.full((DL // 2,), 65536, jnp.int32),
        jnp.zeros(((LP - L2) // 2,), jnp.int32),
    ])
    wvec = jnp.concatenate([half, half]).reshape(4, 128)
    zvec = jnp.zeros((4, 128), jnp.int32)
    c0, c1 = _sc_count(ids_pad, wvec, zvec)

    w01 = jnp.stack([W1[0, :], b1], axis=1)
    b2r = (2.0 * jnp.concatenate([b2, b2])).reshape(1, 2 * D)
    srcp, dstp = _tc_encode(c0.reshape(B, LP), c1.reshape(B, LP), w01, W2, b2r)
    return srcp.reshape(B, SL, D), dstp.reshape(B, DL, D)


# speedup vs baseline: 1.4115x; 1.4115x over previous
import functools

import jax
import jax.numpy as jnp
from jax import lax
from jax.experimental import pallas as pl
from jax.experimental.pallas import tpu as pltpu
from jax.experimental.pallas import tpu_sc as plsc

B, SL, DL, D = 1024, 200, 200, 64
L2 = SL + DL
LP = 512
NC, NS = 2, 16
NW = NC * NS
ROWS_PER_W = B // NW
HSIZE = 100000
RB = 16

_mesh = plsc.VectorSubcoreMesh(core_axis_name="c", subcore_axis_name="s",
                               num_cores=NC, num_subcores=NS)


def _sc_count_body(ids_hbm, wvec_hbm, zvec_hbm, c0_hbm, c1_hbm,
                   hist, ids_v, wvec_v, zvec_v, idx_v, cnt_v, c0_v, c1_v):
    c = lax.axis_index("c")
    s = lax.axis_index("s")
    wid = c * NS + s
    pltpu.sync_copy(wvec_hbm, wvec_v)
    pltpu.sync_copy(zvec_hbm, zvec_v)

    def row_step(r, carry):
        row = wid * ROWS_PER_W + r
        pltpu.sync_copy(ids_hbm.at[row], ids_v)
        for j in range(4):
            for k in range(8):
                sl = pl.ds(k * 16, 16)
                idx_v[j, sl] = ids_v[j, sl] + s * HSIZE
        for j in range(4):
            pltpu.sync_copy(zvec_v.at[j], hist.at[idx_v.at[j]])
        for j in range(4):
            pltpu.sync_copy(wvec_v.at[j], hist.at[idx_v.at[j]], add=True)
        for j in range(4):
            pltpu.sync_copy(hist.at[idx_v.at[j]], cnt_v.at[j])
        for j in range(4):
            for k in range(8):
                sl = pl.ds(k * 16, 16)
                ids16 = ids_v[j, sl]
                cnt16 = cnt_v[j, sl]
                nz = ids16 != 0
                c0_v[j, sl] = jnp.where(nz, cnt16 & 0xFFFF, 0).astype(jnp.float32)
                c1_v[j, sl] = jnp.where(nz, cnt16 >> 16, 0).astype(jnp.float32)
        pltpu.sync_copy(c0_v, c0_hbm.at[row])
        pltpu.sync_copy(c1_v, c1_hbm.at[row])
        return carry

    lax.fori_loop(0, ROWS_PER_W, row_step, 0)


@functools.partial(
    pl.kernel,
    out_type=(
        jax.ShapeDtypeStruct((B, 4, 128), jnp.float32),
        jax.ShapeDtypeStruct((B, 4, 128), jnp.float32),
    ),
    mesh=_mesh,
    scratch_types=[
        pltpu.VMEM_SHARED((NS * HSIZE,), jnp.int32),
        pltpu.VMEM((4, 128), jnp.int32),
        pltpu.VMEM((4, 128), jnp.int32),
        pltpu.VMEM((4, 128), jnp.int32),
        pltpu.VMEM((4, 128), jnp.int32),
        pltpu.VMEM((4, 128), jnp.int32),
        pltpu.VMEM((4, 128), jnp.float32),
        pltpu.VMEM((4, 128), jnp.float32),
    ],
)
def _sc_count(*args):
    _sc_count_body(*args)


def _tc_encode_body(c0_ref, c1_ref, w01_ref, w2_ref, b2r_ref,
                    src_out, dst_out):
    w01 = w01_ref[...]
    w2 = w2_ref[...]
    b2r = b2r_ref[...]
    one_row = jnp.ones((1, LP), jnp.float32)
    for b in range(RB):
        s0 = jnp.concatenate([c0_ref[b:b + 1, :], one_row], axis=0)
        s1 = jnp.concatenate([c1_ref[b:b + 1, :], one_row], axis=0)
        p0t = jnp.dot(w01, s0, preferred_element_type=jnp.float32)
        p1t = jnp.dot(w01, s1, preferred_element_type=jnp.float32)
        ht = jnp.maximum(p0t, 0.0) + jnp.maximum(p1t, 0.0)
        feat = lax.dot_general(ht, w2, (((0,), (0,)), ((), ())),
                               preferred_element_type=jnp.float32)
        feat = feat + b2r
        src_out[b, :, :] = feat[:SL, :]
        dst_out[b, :, :] = feat[SL:L2, :]


def _tc_encode(c0, c1, w01, w2, b2r):
    return pl.pallas_call(
        _tc_encode_body,
        grid=(B // RB,),
        in_specs=[
            pl.BlockSpec((RB, LP), lambda i: (i, 0)),
            pl.BlockSpec((RB, LP), lambda i: (i, 0)),
            pl.BlockSpec((D, 2), lambda i: (0, 0)),
            pl.BlockSpec((D, D), lambda i: (0, 0)),
            pl.BlockSpec((1, D), lambda i: (0, 0)),
        ],
        out_specs=[
            pl.BlockSpec((RB, SL, D), lambda i: (i, 0, 0)),
            pl.BlockSpec((RB, DL, D), lambda i: (i, 0, 0)),
        ],
        out_shape=[
            jax.ShapeDtypeStruct((B, SL, D), jnp.float32),
            jax.ShapeDtypeStruct((B, DL, D), jnp.float32),
        ],
        compiler_params=pltpu.CompilerParams(
            dimension_semantics=("parallel",)),
    )(c0, c1, w01, w2, b2r)


@jax.jit
def kernel(src_ids, dst_ids, W1, b1, W2, b2):
    ids = jnp.concatenate([src_ids.astype(jnp.int32),
                           dst_ids.astype(jnp.int32)], axis=1)
    ids_pad = jnp.pad(ids, ((0, 0), (0, LP - L2))).reshape(B, 4, 128)
    wvec = jnp.concatenate([
        jnp.full((SL,), 1, jnp.int32),
        jnp.full((DL,), 65536, jnp.int32),
        jnp.zeros((LP - L2,), jnp.int32),
    ]).reshape(4, 128)
    zvec = jnp.zeros((4, 128), jnp.int32)
    c0, c1 = _sc_count(ids_pad, wvec, zvec)

    w01 = jnp.stack([W1[0, :], b1], axis=1)
    b2r = (2.0 * b2).reshape(1, D)
    return _tc_encode(c0.reshape(B, LP), c1.reshape(B, LP), w01, W2, b2r)
